# baseline (speedup 1.0000x reference)
import functools

import jax
import jax.numpy as jnp
from jax import lax
from jax.experimental import pallas as pl
from jax.experimental.pallas import tpu as pltpu
from jax.experimental.pallas import tpu_sc as plsc

VOCAB_N = 100000
EMB_D = 64
BATCH_N = 4096
CTX_L = 20

NUM_CORES = 2
NUM_SUBCORES = 16
NUM_WORKERS = NUM_CORES * NUM_SUBCORES
DIMS_PER_W = EMB_D // NUM_WORKERS
LANES = 16
BCHUNK = 512

TILE_V = 1024
CB = 1024
B_SPLIT = 1024


def _sc_gather_mean_t(idx_t, emb_t):
    bn = idx_t.shape[1]
    n_bchunks = bn // BCHUNK
    mesh = plsc.VectorSubcoreMesh(core_axis_name="c", subcore_axis_name="s")

    @functools.partial(
        pl.kernel,
        mesh=mesh,
        compiler_params=pltpu.CompilerParams(
            use_tc_tiling_on_sc=False, needs_layout_passes=False),
        out_type=jax.ShapeDtypeStruct((EMB_D, bn), jnp.float32),
        scratch_types=[
            pltpu.VMEM((VOCAB_N,), jnp.float32),
            pltpu.VMEM((2 * CTX_L, BCHUNK), jnp.int32),
            pltpu.VMEM((BCHUNK,), jnp.float32),
            pltpu.SemaphoreType.DMA,
            pltpu.SemaphoreType.DMA,
            pltpu.SemaphoreType.DMA,
        ],
    )
    def run(idx_hbm, emb_hbm, out_hbm, row_v, idx_v, avg_v, sem_r, sem_i, sem_o):
        wid = lax.axis_index("s") * NUM_CORES + lax.axis_index("c")

        for d in range(DIMS_PER_W):
            e = wid * DIMS_PER_W + d
            row_cp = pltpu.async_copy(emb_hbm.at[e], row_v, sem_r)
            idx_cp = pltpu.async_copy(
                idx_hbm.at[:, pl.ds(0, BCHUNK)],
                idx_v.at[pl.ds(0, CTX_L)], sem_i)
            row_cp.wait()

            for q in range(n_bchunks):
                idx_cp.wait()
                if q + 1 < n_bchunks:
                    idx_cp = pltpu.async_copy(
                        idx_hbm.at[:, pl.ds((q + 1) * BCHUNK, BCHUNK)],
                        idx_v.at[pl.ds(((q + 1) % 2) * CTX_L, CTX_L)], sem_i)

                def chunk_body(i, carry, q=q):
                    b0 = i * LANES
                    jbase = (q % 2) * CTX_L

                    def ctx_step(j, acc):
                        iv = idx_v[jbase + j, pl.ds(b0, LANES)]
                        return acc + plsc.load_gather(row_v, [iv])

                    acc = lax.fori_loop(
                        0, CTX_L, ctx_step, jnp.zeros((LANES,), jnp.float32))
                    avg_v[pl.ds(b0, LANES)] = acc * (1.0 / CTX_L)
                    return carry

                lax.fori_loop(0, BCHUNK // LANES, chunk_body, 0)
                pltpu.async_copy(
                    avg_v, out_hbm.at[e, pl.ds(q * BCHUNK, BCHUNK)], sem_o
                ).wait()

    return run(idx_t, emb_t)


def _tc_body(wt_ref, avg_ref, b_ref, out_ref):
    out_ref[...] = lax.dot_general(
        wt_ref[...], avg_ref[...],
        (((0,), (0,)), ((), ())),
        preferred_element_type=jnp.float32,
    ) + b_ref[...]


def _tc_matmul_bias_first(avg_t, w_t, b2):
    grid = (B_SPLIT // CB, pl.cdiv(VOCAB_N, TILE_V))
    return pl.pallas_call(
        _tc_body,
        grid=grid,
        in_specs=[
            pl.BlockSpec((EMB_D, TILE_V), lambda i, j: (0, j)),
            pl.BlockSpec((EMB_D, CB), lambda i, j: (0, i)),
            pl.BlockSpec((TILE_V, 1), lambda i, j: (j, 0)),
        ],
        out_specs=pl.BlockSpec((TILE_V, CB), lambda i, j: (j, i)),
        out_shape=jax.ShapeDtypeStruct((VOCAB_N, BATCH_N), jnp.float32),
    )(w_t, avg_t, b2)


def _tc_matmul_bias_rest(avg_t, w_t, b2, prev):
    bn = avg_t.shape[1]
    cb0 = B_SPLIT // CB
    grid = (bn // CB, pl.cdiv(VOCAB_N, TILE_V))

    def body(prev_ref, wt_ref, avg_ref, b_ref, out_ref):
        del prev_ref
        _tc_body(wt_ref, avg_ref, b_ref, out_ref)

    return pl.pallas_call(
        body,
        grid=grid,
        in_specs=[
            pl.BlockSpec(memory_space=pl.ANY),
            pl.BlockSpec((EMB_D, TILE_V), lambda i, j: (0, j)),
            pl.BlockSpec((EMB_D, CB), lambda i, j: (0, i)),
            pl.BlockSpec((TILE_V, 1), lambda i, j: (j, 0)),
        ],
        out_specs=pl.BlockSpec((TILE_V, CB), lambda i, j: (j, cb0 + i)),
        out_shape=jax.ShapeDtypeStruct((VOCAB_N, BATCH_N), jnp.float32),
        input_output_aliases={0: 0},
    )(prev, w_t, avg_t, b2)


def kernel(context_indices, emb_table, W, b):
    idx_t = context_indices.astype(jnp.int32).T
    emb_t = emb_table.T
    w_t = W.T
    b2 = b.reshape(VOCAB_N, 1)
    avg_a = _sc_gather_mean_t(idx_t[:, :B_SPLIT], emb_t)
    avg_b = _sc_gather_mean_t(idx_t[:, B_SPLIT:], emb_t)
    out_a = _tc_matmul_bias_first(avg_a, w_t, b2)
    out_t = _tc_matmul_bias_rest(avg_b, w_t, b2, out_a)
    return out_t.T


# speedup vs baseline: 1.3306x; 1.3306x over previous
import functools

import jax
import jax.numpy as jnp
from jax import lax
from jax.experimental import pallas as pl
from jax.experimental.pallas import tpu as pltpu
from jax.experimental.pallas import tpu_sc as plsc

VOCAB_N = 100000
EMB_D = 64
BATCH_N = 4096
CTX_L = 20

NUM_CORES = 2
NUM_SUBCORES = 16
NUM_WORKERS = NUM_CORES * NUM_SUBCORES
DIMS_PER_W = EMB_D // NUM_WORKERS
LANES = 16
BCHUNK = 512
N_BCHUNKS = BATCH_N // BCHUNK


def _sc_gather_mean_t(idx_t, emb_t):
    mesh = plsc.VectorSubcoreMesh(core_axis_name="c", subcore_axis_name="s")

    @functools.partial(
        pl.kernel,
        mesh=mesh,
        compiler_params=pltpu.CompilerParams(
            use_tc_tiling_on_sc=False, needs_layout_passes=False),
        out_type=jax.ShapeDtypeStruct((EMB_D, BATCH_N), jnp.float32),
        scratch_types=[
            pltpu.VMEM((VOCAB_N,), jnp.float32),
            pltpu.VMEM((2 * CTX_L, BCHUNK), jnp.int32),
            pltpu.VMEM((DIMS_PER_W, BATCH_N), jnp.float32),
            pltpu.SemaphoreType.DMA,
            pltpu.SemaphoreType.DMA,
            pltpu.SemaphoreType.DMA,
        ],
    )
    def run(idx_hbm, emb_hbm, out_hbm, row_v, idx_v, avg_v, sem_r, sem_i, sem_o):
        wid = lax.axis_index("s") * NUM_CORES + lax.axis_index("c")

        out_cp = None
        for d in range(DIMS_PER_W):
            e = wid * DIMS_PER_W + d
            row_cp = pltpu.async_copy(emb_hbm.at[e], row_v, sem_r)
            idx_cp = pltpu.async_copy(
                idx_hbm.at[:, pl.ds(0, BCHUNK)],
                idx_v.at[pl.ds(0, CTX_L)], sem_i)
            row_cp.wait()

            for q in range(N_BCHUNKS):
                idx_cp.wait()
                if q + 1 < N_BCHUNKS:
                    idx_cp = pltpu.async_copy(
                        idx_hbm.at[:, pl.ds((q + 1) * BCHUNK, BCHUNK)],
                        idx_v.at[pl.ds(((q + 1) % 2) * CTX_L, CTX_L)], sem_i)

                def chunk_body(i, carry, q=q, d=d):
                    b0 = i * LANES
                    jbase = (q % 2) * CTX_L

                    def ctx_step(j, acc):
                        iv = idx_v[jbase + j, pl.ds(b0, LANES)]
                        return acc + plsc.load_gather(row_v, [iv])

                    acc = lax.fori_loop(
                        0, CTX_L, ctx_step, jnp.zeros((LANES,), jnp.float32))
                    avg_v[d, pl.ds(q * BCHUNK + b0, LANES)] = acc * (1.0 / CTX_L)
                    return carry

                lax.fori_loop(0, BCHUNK // LANES, chunk_body, 0)

            if out_cp is not None:
                out_cp.wait()
            out_cp = pltpu.async_copy(avg_v.at[d], out_hbm.at[e], sem_o)
        out_cp.wait()

    return run(idx_t, emb_t)


def _tc_matmul_bias_t(avg_t, w_t, b2):
    tile_v = 1024
    grid = (pl.cdiv(VOCAB_N, tile_v),)

    def body(wt_ref, avg_ref, b_ref, out_ref):
        out_ref[...] = lax.dot_general(
            wt_ref[...], avg_ref[...],
            (((0,), (0,)), ((), ())),
            preferred_element_type=jnp.float32,
        ) + b_ref[...]

    return pl.pallas_call(
        body,
        grid=grid,
        in_specs=[
            pl.BlockSpec((EMB_D, tile_v), lambda j: (0, j)),
            pl.BlockSpec((EMB_D, BATCH_N), lambda j: (0, 0)),
            pl.BlockSpec((tile_v, 1), lambda j: (j, 0)),
        ],
        out_specs=pl.BlockSpec((tile_v, BATCH_N), lambda j: (j, 0)),
        out_shape=jax.ShapeDtypeStruct((VOCAB_N, BATCH_N), jnp.float32),
    )(w_t, avg_t, b2)


def kernel(context_indices, emb_table, W, b):
    idx_t = context_indices.astype(jnp.int32).T
    avg_t = _sc_gather_mean_t(idx_t, emb_table.T)
    out_t = _tc_matmul_bias_t(avg_t, W.T, b.reshape(VOCAB_N, 1))
    return out_t.T


# speedup vs baseline: 1.3316x; 1.0007x over previous
import functools

import jax
import jax.numpy as jnp
from jax import lax
from jax.experimental import pallas as pl
from jax.experimental.pallas import tpu as pltpu
from jax.experimental.pallas import tpu_sc as plsc

VOCAB_N = 100000
EMB_D = 64
BATCH_N = 4096
CTX_L = 20

NUM_CORES = 2
NUM_SUBCORES = 16
NUM_WORKERS = NUM_CORES * NUM_SUBCORES
DIMS_PER_W = EMB_D // NUM_WORKERS
LANES = 16
BCHUNK = 512
N_BCHUNKS = BATCH_N // BCHUNK


def _sc_gather_mean_t(idx_t, emb_t):
    mesh = plsc.VectorSubcoreMesh(core_axis_name="c", subcore_axis_name="s")

    @functools.partial(
        pl.kernel,
        mesh=mesh,
        compiler_params=pltpu.CompilerParams(
            use_tc_tiling_on_sc=False, needs_layout_passes=False),
        out_type=jax.ShapeDtypeStruct((EMB_D, BATCH_N), jnp.float32),
        scratch_types=[
            pltpu.VMEM((VOCAB_N,), jnp.float32),
            pltpu.VMEM((2 * CTX_L, BCHUNK), jnp.int32),
            pltpu.VMEM((DIMS_PER_W, BATCH_N), jnp.float32),
            pltpu.SemaphoreType.DMA,
            pltpu.SemaphoreType.DMA,
            pltpu.SemaphoreType.DMA,
        ],
    )
    def run(idx_hbm, emb_hbm, out_hbm, row_v, idx_v, avg_v, sem_r, sem_i, sem_o):
        wid = lax.axis_index("s") * NUM_CORES + lax.axis_index("c")

        out_cp = None
        for d in range(DIMS_PER_W):
            e = wid * DIMS_PER_W + d
            row_cp = pltpu.async_copy(emb_hbm.at[e], row_v, sem_r)
            idx_cp = pltpu.async_copy(
                idx_hbm.at[:, pl.ds(0, BCHUNK)],
                idx_v.at[pl.ds(0, CTX_L)], sem_i)
            row_cp.wait()

            for q in range(N_BCHUNKS):
                idx_cp.wait()
                if q + 1 < N_BCHUNKS:
                    idx_cp = pltpu.async_copy(
                        idx_hbm.at[:, pl.ds((q + 1) * BCHUNK, BCHUNK)],
                        idx_v.at[pl.ds(((q + 1) % 2) * CTX_L, CTX_L)], sem_i)

                def chunk_body(i, carry, q=q, d=d):
                    b0 = i * LANES
                    jbase = (q % 2) * CTX_L

                    def ctx_step(j, acc):
                        iv = idx_v[jbase + j, pl.ds(b0, LANES)]
                        return acc + plsc.load_gather(row_v, [iv])

                    acc = lax.fori_loop(
                        0, CTX_L, ctx_step, jnp.zeros((LANES,), jnp.float32))
                    avg_v[d, pl.ds(q * BCHUNK + b0, LANES)] = acc * (1.0 / CTX_L)
                    return carry

                lax.fori_loop(0, BCHUNK // LANES, chunk_body, 0)

            if out_cp is not None:
                out_cp.wait()
            out_cp = pltpu.async_copy(avg_v.at[d], out_hbm.at[e], sem_o)
        out_cp.wait()

    return run(idx_t, emb_t)


def _tc_matmul_bias_t(avg_t, w_t, b2):
    tile_v = 1024
    grid = (pl.cdiv(VOCAB_N, tile_v),)

    def body(wt_ref, avg_ref, b_ref, out_ref):
        out_ref[...] = lax.dot_general(
            wt_ref[...], avg_ref[...],
            (((0,), (0,)), ((), ())),
            preferred_element_type=jnp.float32,
        ) + b_ref[...]

    return pl.pallas_call(
        body,
        grid=grid,
        compiler_params=pltpu.CompilerParams(
            dimension_semantics=("parallel",)),
        in_specs=[
            pl.BlockSpec((EMB_D, tile_v), lambda j: (0, j)),
            pl.BlockSpec((EMB_D, BATCH_N), lambda j: (0, 0)),
            pl.BlockSpec((tile_v, 1), lambda j: (j, 0)),
        ],
        out_specs=pl.BlockSpec((tile_v, BATCH_N), lambda j: (j, 0)),
        out_shape=jax.ShapeDtypeStruct((VOCAB_N, BATCH_N), jnp.float32),
    )(w_t, avg_t, b2)


def kernel(context_indices, emb_table, W, b):
    idx_t = context_indices.astype(jnp.int32).T
    avg_t = _sc_gather_mean_t(idx_t, emb_table.T)
    out_t = _tc_matmul_bias_t(avg_t, W.T, b.reshape(VOCAB_N, 1))
    return out_t.T


# speedup vs baseline: 1.3329x; 1.0010x over previous
import functools

import jax
import jax.numpy as jnp
from jax import lax
from jax.experimental import pallas as pl
from jax.experimental.pallas import tpu as pltpu
from jax.experimental.pallas import tpu_sc as plsc

VOCAB_N = 100000
EMB_D = 64
BATCH_N = 4096
CTX_L = 20

NUM_CORES = 2
NUM_SUBCORES = 16
NUM_WORKERS = NUM_CORES * NUM_SUBCORES
DIMS_PER_W = EMB_D // NUM_WORKERS
LANES = 16
BCHUNK = 512
N_BCHUNKS = BATCH_N // BCHUNK


def _sc_gather_mean_t(idx_t, emb_t):
    mesh = plsc.VectorSubcoreMesh(core_axis_name="c", subcore_axis_name="s")

    @functools.partial(
        pl.kernel,
        mesh=mesh,
        compiler_params=pltpu.CompilerParams(
            use_tc_tiling_on_sc=False, needs_layout_passes=False),
        out_type=jax.ShapeDtypeStruct((EMB_D, BATCH_N), jnp.float32),
        scratch_types=[
            pltpu.VMEM((VOCAB_N,), jnp.float32),
            pltpu.VMEM((2 * CTX_L, BCHUNK), jnp.int32),
            pltpu.VMEM((DIMS_PER_W, BATCH_N), jnp.float32),
            pltpu.SemaphoreType.DMA,
            pltpu.SemaphoreType.DMA,
            pltpu.SemaphoreType.DMA,
        ],
    )
    def run(idx_hbm, emb_hbm, out_hbm, row_v, idx_v, avg_v, sem_r, sem_i, sem_o):
        wid = lax.axis_index("s") * NUM_CORES + lax.axis_index("c")

        out_cp = None
        for d in range(DIMS_PER_W):
            e = wid * DIMS_PER_W + d
            row_cp = pltpu.async_copy(emb_hbm.at[e], row_v, sem_r)
            idx_cp = pltpu.async_copy(
                idx_hbm.at[:, pl.ds(0, BCHUNK)],
                idx_v.at[pl.ds(0, CTX_L)], sem_i)
            row_cp.wait()

            for q in range(N_BCHUNKS):
                idx_cp.wait()
                if q + 1 < N_BCHUNKS:
                    idx_cp = pltpu.async_copy(
                        idx_hbm.at[:, pl.ds((q + 1) * BCHUNK, BCHUNK)],
                        idx_v.at[pl.ds(((q + 1) % 2) * CTX_L, CTX_L)], sem_i)

                def chunk_body(i, carry, q=q, d=d):
                    b0 = i * LANES
                    jbase = (q % 2) * CTX_L

                    def ctx_step(j, acc):
                        iv = idx_v[jbase + j, pl.ds(b0, LANES)]
                        return acc + plsc.load_gather(row_v, [iv])

                    acc = lax.fori_loop(
                        0, CTX_L, ctx_step, jnp.zeros((LANES,), jnp.float32))
                    avg_v[d, pl.ds(q * BCHUNK + b0, LANES)] = acc * (1.0 / CTX_L)
                    return carry

                lax.fori_loop(0, BCHUNK // LANES, chunk_body, 0)

            if out_cp is not None:
                out_cp.wait()
            out_cp = pltpu.async_copy(avg_v.at[d], out_hbm.at[e], sem_o)
        out_cp.wait()

    return run(idx_t, emb_t)


def _tc_matmul_bias_t(avg_t, w_t, b2):
    tile_v = 1536
    grid = (pl.cdiv(VOCAB_N, tile_v),)

    def body(wt_ref, avg_ref, b_ref, out_ref):
        out_ref[...] = lax.dot_general(
            wt_ref[...], avg_ref[...],
            (((0,), (0,)), ((), ())),
            preferred_element_type=jnp.float32,
        ) + b_ref[...]

    return pl.pallas_call(
        body,
        grid=grid,
        compiler_params=pltpu.CompilerParams(
            dimension_semantics=("parallel",)),
        in_specs=[
            pl.BlockSpec((EMB_D, tile_v), lambda j: (0, j)),
            pl.BlockSpec((EMB_D, BATCH_N), lambda j: (0, 0)),
            pl.BlockSpec((tile_v, 1), lambda j: (j, 0)),
        ],
        out_specs=pl.BlockSpec((tile_v, BATCH_N), lambda j: (j, 0)),
        out_shape=jax.ShapeDtypeStruct((VOCAB_N, BATCH_N), jnp.float32),
    )(w_t, avg_t, b2)


def kernel(context_indices, emb_table, W, b):
    idx_t = context_indices.astype(jnp.int32).T
    avg_t = _sc_gather_mean_t(idx_t, emb_table.T)
    out_t = _tc_matmul_bias_t(avg_t, W.T, b.reshape(VOCAB_N, 1))
    return out_t.T
